# baseline (speedup 1.0000x reference)
import functools

import jax
import jax.numpy as jnp
from jax import lax
from jax.experimental import pallas as pl
from jax.experimental.pallas import tpu as pltpu
from jax.experimental.pallas import tpu_sc as plsc

INTERPRET = False

N = 10000
E = 320000
BE = 1600
BN = 1000


def _leaky(h):
    return jnp.where(h >= 0, h, 0.01 * h)


def _ln(h):
    mu = jnp.mean(h, axis=-1, keepdims=True)
    var = jnp.mean((h - mu) ** 2, axis=-1, keepdims=True)
    return (h - mu) * jax.lax.rsqrt(var + 1e-5)


def _vspec(shape):
    return pl.BlockSpec(shape, lambda i: (0,) * len(shape))


def _enc_edges_body(e_in, wenc, benc, we0, bcore, e_enc, p0):
    ee = _leaky(jnp.dot(e_in[...], wenc[...]) + benc[...])
    e_enc[...] = ee
    p0[...] = jnp.dot(ee, we0[...]) + bcore[...]


def _enc_edges(e_in, wenc, benc, we0, bcore):
    g = E // BE
    return pl.pallas_call(
        _enc_edges_body,
        grid=(g,),
        in_specs=[
            pl.BlockSpec((BE, 16), lambda i: (i, 0)),
            _vspec((16, 128)),
            _vspec((1, 128)),
            _vspec((128, 128)),
            _vspec((1, 128)),
        ],
        out_specs=[
            pl.BlockSpec((BE, 128), lambda i: (i, 0)),
            pl.BlockSpec((BE, 128), lambda i: (i, 0)),
        ],
        out_shape=[
            jax.ShapeDtypeStruct((E + 128, 128), jnp.float32),
            jax.ShapeDtypeStruct((E + 128, 128), jnp.float32),
        ],
        interpret=INTERPRET,
    )(e_in, wenc, benc, we0, bcore)


def _enc_nodes_body(x_in, wenc, benc, ws0, wd0, wx0, bx, x_enc, xs0, xd0, xc0):
    xe = _leaky(jnp.dot(x_in[...], wenc[...]) + benc[...])
    x_enc[...] = xe
    xs0[...] = jnp.dot(xe, ws0[...])
    xd0[...] = jnp.dot(xe, wd0[...])
    xc0[...] = jnp.dot(xe, wx0[...]) + bx[...]


def _enc_nodes(x_in, wenc, benc, ws0, wd0, wx0, bx):
    g = N // BN
    return pl.pallas_call(
        _enc_nodes_body,
        grid=(g,),
        in_specs=[
            pl.BlockSpec((BN, 128), lambda i: (i, 0)),
            _vspec((128, 128)),
            _vspec((1, 128)),
            _vspec((128, 128)),
            _vspec((128, 128)),
            _vspec((128, 128)),
            _vspec((1, 128)),
        ],
        out_specs=[pl.BlockSpec((BN, 128), lambda i: (i, 0))] * 4,
        out_shape=[jax.ShapeDtypeStruct((N, 128), jnp.float32)] * 4,
        interpret=INTERPRET,
    )(x_in, wenc, benc, ws0, wd0, wx0, bx)


def _g_enc_body(g_in, w, b, out):
    out[...] = _leaky(jnp.dot(g_in[...], w[...]) + b[...])


def _g_enc(g_in, wpad, bpad):
    return pl.pallas_call(
        _g_enc_body,
        grid=(1,),
        in_specs=[_vspec((1, 16)), _vspec((16, 128)), _vspec((1, 128))],
        out_specs=_vspec((1, 128)),
        out_shape=jax.ShapeDtypeStruct((1, 128), jnp.float32),
        interpret=INTERPRET,
    )(g_in, wpad, bpad)


def _node_prep_body(x, ws1, wd1, xs0, xd0, xs, xd):
    xv = x[...]
    xs[...] = jnp.dot(xv, ws1[...]) + xs0[...]
    xd[...] = jnp.dot(xv, wd1[...]) + xd0[...]


def _node_prep(x, ws1, wd1, xs0, xd0):
    g = N // BN
    return pl.pallas_call(
        _node_prep_body,
        grid=(g,),
        in_specs=[
            pl.BlockSpec((BN, 128), lambda i: (i, 0)),
            _vspec((128, 128)),
            _vspec((128, 128)),
            pl.BlockSpec((BN, 128), lambda i: (i, 0)),
            pl.BlockSpec((BN, 128), lambda i: (i, 0)),
        ],
        out_specs=[pl.BlockSpec((BN, 128), lambda i: (i, 0))] * 2,
        out_shape=[jax.ShapeDtypeStruct((N, 128), jnp.float32)] * 2,
        interpret=INTERPRET,
    )(x, ws1, wd1, xs0, xd0)


def _edge_core_body(e, p0, xg, g0b, gb, wg2, we1, e_new, esum, emax, emin):
    i = pl.program_id(0)
    grow = g0b[...] * wg2[0:1, :] + gb[...] * wg2[1:2, :]
    h = jnp.dot(e[...], we1[...]) + p0[...] + xg[...] + grow
    y = _ln(_leaky(h))
    e_new[...] = y
    bs = jnp.sum(y, axis=0, keepdims=True)
    bmx = jnp.max(y, axis=0, keepdims=True)
    bmn = jnp.min(y, axis=0, keepdims=True)

    @pl.when(i == 0)
    def _():
        esum[...] = bs
        emax[...] = bmx
        emin[...] = bmn

    @pl.when(i != 0)
    def _():
        esum[...] += bs
        emax[...] = jnp.maximum(emax[...], bmx)
        emin[...] = jnp.minimum(emin[...], bmn)


def _edge_core(e, p0, xg, g0b, gb, wg2, we1):
    g = E // BE
    return pl.pallas_call(
        _edge_core_body,
        grid=(g,),
        in_specs=[
            pl.BlockSpec((BE, 128), lambda i: (i, 0)),
            pl.BlockSpec((BE, 128), lambda i: (i, 0)),
            pl.BlockSpec((BE, 128), lambda i: (i, 0)),
            _vspec((1, 128)),
            _vspec((1, 128)),
            _vspec((2, 128)),
            _vspec((128, 128)),
        ],
        out_specs=[
            pl.BlockSpec((BE, 128), lambda i: (i, 0)),
            _vspec((1, 128)),
            _vspec((1, 128)),
            _vspec((1, 128)),
        ],
        out_shape=[
            jax.ShapeDtypeStruct((E + 128, 128), jnp.float32),
            jax.ShapeDtypeStruct((1, 128), jnp.float32),
            jax.ShapeDtypeStruct((1, 128), jnp.float32),
            jax.ShapeDtypeStruct((1, 128), jnp.float32),
        ],
        interpret=INTERPRET,
    )(e, p0, xg, g0b, gb, wg2, we1)


def _node_core_body(x, xc0, nsum, nmax, nmin, cnt, a1, a2, a3, a4, bagg, x1w,
                    xaw, g0b, gb, wgx, x_new, xsum, xmax, xmin):
    i = pl.program_id(0)
    c = cnt[...]
    has = c > 0.0
    mx = jnp.where(has, nmax[...], 0.0)
    mn = jnp.where(has, nmin[...], 0.0)
    s = jnp.where(has, nsum[...], 0.0)
    mean = s / jnp.maximum(c, 1.0)
    agg = _leaky(
        jnp.dot(s, a1[...]) + jnp.dot(mx, a2[...]) + jnp.dot(mean, a3[...])
        + jnp.dot(mn, a4[...]) + bagg[...]
    )
    grow = g0b[...] * wgx[0:1, :] + gb[...] * wgx[1:2, :]
    h = jnp.dot(x[...], x1w[...]) + xc0[...] + jnp.dot(agg, xaw[...]) + grow
    y = _ln(_leaky(h))
    x_new[...] = y
    bs = jnp.sum(y, axis=0, keepdims=True)
    bmx = jnp.max(y, axis=0, keepdims=True)
    bmn = jnp.min(y, axis=0, keepdims=True)

    @pl.when(i == 0)
    def _():
        xsum[...] = bs
        xmax[...] = bmx
        xmin[...] = bmn

    @pl.when(i != 0)
    def _():
        xsum[...] += bs
        xmax[...] = jnp.maximum(xmax[...], bmx)
        xmin[...] = jnp.minimum(xmin[...], bmn)


def _node_core(x, xc0, nsum, nmax, nmin, cnt, a1, a2, a3, a4, bagg, x1w, xaw,
               g0b, gb, wgx):
    g = N // BN
    bspec = pl.BlockSpec((BN, 128), lambda i: (i, 0))
    return pl.pallas_call(
        _node_core_body,
        grid=(g,),
        in_specs=[
            bspec, bspec, bspec, bspec, bspec,
            pl.BlockSpec((BN, 1), lambda i: (i, 0)),
            _vspec((128, 128)), _vspec((128, 128)), _vspec((128, 128)),
            _vspec((128, 128)), _vspec((1, 128)),
            _vspec((128, 128)), _vspec((128, 128)),
            _vspec((1, 128)), _vspec((1, 128)), _vspec((2, 128)),
        ],
        out_specs=[
            bspec,
            _vspec((1, 128)), _vspec((1, 128)), _vspec((1, 128)),
        ],
        out_shape=[
            jax.ShapeDtypeStruct((N, 128), jnp.float32),
            jax.ShapeDtypeStruct((1, 128), jnp.float32),
            jax.ShapeDtypeStruct((1, 128), jnp.float32),
            jax.ShapeDtypeStruct((1, 128), jnp.float32),
        ],
        interpret=INTERPRET,
    )(x, xc0, nsum, nmax, nmin, cnt, a1, a2, a3, a4, bagg, x1w, xaw, g0b, gb,
      wgx)


def _global_body(esum, emax, emin, xsum, xmax, xmin, g0b, gb, ge, bge, gn, bgn,
                 wcg, gb_new):
    es = esum[...]
    eagg = _leaky(
        jnp.dot(es, ge[0:128, :]) + jnp.dot(emax[...], ge[128:256, :])
        + jnp.dot(es * (1.0 / E), ge[256:384, :])
        + jnp.dot(emin[...], ge[384:512, :]) + bge[...]
    )
    xs = xsum[...]
    nagg = _leaky(
        jnp.dot(xs, gn[0:128, :]) + jnp.dot(xmax[...], gn[128:256, :])
        + jnp.dot(xs * (1.0 / N), gn[256:384, :])
        + jnp.dot(xmin[...], gn[384:512, :]) + bgn[...]
    )
    h = (
        g0b[0:1, 0:1] * wcg[0:1, 0:1] + gb[0:1, 0:1] * wcg[0:1, 1:2]
        + wcg[0:1, 2:3]
        + jnp.sum(eagg * wcg[1:2, :], axis=-1, keepdims=True)
        + jnp.sum(nagg * wcg[2:3, :], axis=-1, keepdims=True)
    )
    y = _leaky(h)
    gb_new[...] = jnp.broadcast_to((y - y) * jax.lax.rsqrt(1e-5),
                                   gb_new.shape)


def _global_core(esum, emax, emin, xsum, xmax, xmin, g0b, gb, ge, bge, gn, bgn,
                 wcg):
    return pl.pallas_call(
        _global_body,
        grid=(1,),
        in_specs=[
            _vspec((1, 128)), _vspec((1, 128)), _vspec((1, 128)),
            _vspec((1, 128)), _vspec((1, 128)), _vspec((1, 128)),
            _vspec((1, 128)), _vspec((1, 128)),
            _vspec((512, 128)), _vspec((1, 128)),
            _vspec((512, 128)), _vspec((1, 128)),
            _vspec((4, 128)),
        ],
        out_specs=_vspec((1, 128)),
        out_shape=jax.ShapeDtypeStruct((1, 128), jnp.float32),
        interpret=INTERPRET,
    )(esum, emax, emin, xsum, xmax, xmin, g0b, gb, ge, bge, gn, bgn, wcg)


def _dec_body(ow, z, d1, b1, d2, b2, wout, bout, out):
    h = _leaky(jnp.dot(z[...], d1[...]) + b1[...])
    h = _leaky(jnp.dot(h, d2[...]) + b2[...])
    val = jnp.sum(h * wout[...], axis=-1, keepdims=True) + bout[0:1, 0:1]
    out[...] = jnp.broadcast_to(val, out.shape)


def _decode(z, d1, b1, d2, b2, wout, bout, total, blk, ow=1):
    g = total // blk
    return pl.pallas_call(
        functools.partial(_dec_body, ow),
        grid=(g,),
        in_specs=[
            pl.BlockSpec((blk, 128), lambda i: (i, 0)),
            _vspec((128, 128)), _vspec((1, 128)),
            _vspec((128, 128)), _vspec((1, 128)),
            _vspec((1, 128)), _vspec((1, 128)),
        ],
        out_specs=pl.BlockSpec((blk, ow), lambda i: (i, 0)),
        out_shape=jax.ShapeDtypeStruct((total, ow), jnp.float32),
        interpret=INTERPRET,
    )(z, d1, b1, d2, b2, wout, bout)


def _dec_g_body(gb, wpack, out):
    h = _leaky(gb[...] * wpack[0:1, 0:1] + wpack[0:1, 1:2])
    out[...] = h * wpack[0:1, 2:3] + wpack[0:1, 3:4]


def _dec_g(gb, wpack):
    return pl.pallas_call(
        _dec_g_body,
        grid=(1,),
        in_specs=[_vspec((1, 128)), _vspec((1, 128))],
        out_specs=_vspec((1, 128)),
        out_shape=jax.ShapeDtypeStruct((1, 128), jnp.float32),
        interpret=INTERPRET,
    )(gb, wpack)


_NW = 32
_SPAN = E // _NW
_C = 80
_NCH = _SPAN // _C
_NB = 5


def _sc_gather_body(xs_hbm, xd_hbm, src_hbm, dst_hbm, out_hbm, src_v, dst_v,
                    *rest):
    bufs = rest[0:_NB]
    g1s = rest[_NB:2 * _NB]
    g2s = rest[2 * _NB:3 * _NB]
    wbs = rest[3 * _NB:4 * _NB]
    wid = lax.axis_index("s") * 2 + lax.axis_index("c")
    base = wid * _SPAN
    pltpu.sync_copy(src_hbm.at[pl.ds(base, _SPAN)], src_v)
    pltpu.sync_copy(dst_hbm.at[pl.ds(base, _SPAN)], dst_v)

    def g1_start(c, b):
        pltpu.async_copy(xs_hbm.at[src_v.at[pl.ds(c * _C, _C)]], bufs[b],
                         g1s[b])

    def g1_wait(b):
        pltpu.make_async_copy(xs_hbm.at[src_v.at[pl.ds(0, _C)]], bufs[b],
                              g1s[b]).wait()

    for b in range(_NB):
        g1_start(b, b)

    def round_body(it, _):
        c0 = it * _NB
        for b in range(_NB):
            g1_wait(b)
            pltpu.async_copy(xd_hbm.at[dst_v.at[pl.ds((c0 + b) * _C, _C)]],
                             bufs[b], g2s[b], add=True)
        for b in range(_NB):
            pltpu.make_async_copy(xd_hbm.at[dst_v.at[pl.ds(0, _C)]], bufs[b],
                                  g2s[b]).wait()
            pltpu.async_copy(
                bufs[b], out_hbm.at[pl.ds(base + (c0 + b) * _C, _C), :],
                wbs[b])
        for b in range(_NB):
            nxt = c0 + _NB + b

            @pl.when(nxt < _NCH)
            def _():
                pltpu.make_async_copy(
                    bufs[b], out_hbm.at[pl.ds(base, _C), :], wbs[b]).wait()
                g1_start(nxt, b)
        return 0

    lax.fori_loop(0, _NCH // _NB, round_body, 0)
    for b in range(_NB):
        pltpu.make_async_copy(bufs[b], out_hbm.at[pl.ds(base, _C), :],
                              wbs[b]).wait()


@functools.cache
def _sc_gather():
    return pl.kernel(
        _sc_gather_body,
        out_type=jax.ShapeDtypeStruct((E, 128), jnp.float32),
        mesh=plsc.VectorSubcoreMesh(core_axis_name="c", subcore_axis_name="s"),
        scratch_types=(
            [pltpu.VMEM((_SPAN,), jnp.int32)] * 2
            + [pltpu.VMEM((_C, 128), jnp.float32)] * _NB
            + [pltpu.SemaphoreType.DMA] * (3 * _NB)
        ),
    )


def _gather_xg(xs, xd, src, dst):
    return _sc_gather()(xs, xd, src, dst)


def _sc_permute_body(C, D, table_hbm, idx_hbm, out_hbm, idx_v, *rest):
    nch = _SPAN // C
    nb = 5
    bufs = rest[0:nb]
    g1s = rest[nb:2 * nb]
    wbs = rest[2 * nb:3 * nb]
    wid = lax.axis_index("s") * 2 + lax.axis_index("c")
    base = wid * _SPAN
    pltpu.sync_copy(idx_hbm.at[pl.ds(base, _SPAN)], idx_v)

    def g1_start(c, b):
        pltpu.async_copy(table_hbm.at[idx_v.at[pl.ds(c * C, C)]], bufs[b],
                         g1s[b])

    for b in range(nb):
        g1_start(b, b)

    def round_body(it, _):
        c0 = it * nb
        for b in range(nb):
            pltpu.make_async_copy(table_hbm.at[idx_v.at[pl.ds(0, C)]],
                                  bufs[b], g1s[b]).wait()
            pltpu.async_copy(bufs[b],
                             out_hbm.at[pl.ds(base + (c0 + b) * C, C), :],
                             wbs[b])
        for b in range(nb):
            nxt = c0 + nb + b

            @pl.when(nxt < nch)
            def _():
                pltpu.make_async_copy(bufs[b], out_hbm.at[pl.ds(base, C), :],
                                      wbs[b]).wait()
                g1_start(nxt, b)
        return 0

    lax.fori_loop(0, nch // nb, round_body, 0)
    for b in range(nb):
        pltpu.make_async_copy(bufs[b], out_hbm.at[pl.ds(base, C), :],
                              wbs[b]).wait()


@functools.cache
def _sc_permute():
    C = 80
    nb = 5
    D = 128
    return pl.kernel(
        functools.partial(_sc_permute_body, C, D),
        out_type=jax.ShapeDtypeStruct((E + 128, D), jnp.float32),
        mesh=plsc.VectorSubcoreMesh(core_axis_name="c", subcore_axis_name="s"),
        scratch_types=(
            [pltpu.VMEM((_SPAN,), jnp.int32)]
            + [pltpu.VMEM((C, D), jnp.float32)] * nb
            + [pltpu.SemaphoreType.DMA] * (2 * nb)
        ),
    )


_RN = 160
_NR = 64
_NP = _NR * _RN
_C2 = 128
_NEG = -3.0e38
_POS = 3.0e38


def _sc_segred_body(enf_hbm, sdst_hbm, bnd_hbm,
                    osum_hbm, omax_hbm, omin_hbm,
                    bnd_v, sdst_v, rowf, acc_s, acc_x, acc_n):
    wid = lax.axis_index("s") * 2 + lax.axis_index("c")
    pltpu.sync_copy(bnd_hbm, bnd_v.at[pl.ds(0, 8 * _NW)])
    bvec = bnd_v[pl.ds(wid * 8, 16)]
    for j in range(2):
        r = wid * 2 + j
        lo_e = pl.multiple_of(bvec[2 * j] & ~7, 8)
        hi_e = bvec[2 * j + 1]
        node_lo = r * _RN
        node_hi = jnp.minimum(node_lo + _RN, N)
        nch = lax.shift_right_logical(hi_e - lo_e + (_C2 - 1), 7)

        def flush(prev, svec, xvec, nvec):
            in_r = jnp.logical_and(prev >= node_lo, prev < node_hi)

            @pl.when(in_r)
            def _():
                off = (prev - node_lo) * 128
                for k in range(8):
                    acc_s[pl.ds(off + 16 * k, 16)] = svec[k]
                    acc_x[pl.ds(off + 16 * k, 16)] = xvec[k]
                    acc_n[pl.ds(off + 16 * k, 16)] = nvec[k]

        def chunk_body(ci, carry):
            pos = pl.multiple_of(lo_e + ci * _C2, 8)
            pltpu.sync_copy(sdst_hbm.at[pl.ds(pos, _C2)],
                            sdst_v.at[pl.ds(0, _C2)])
            pltpu.sync_copy(enf_hbm.at[pl.ds(pos * 128, _C2 * 128)], rowf)

            def group_body(gi, ec):
                dvec = sdst_v[pl.ds(gi * 8, 16)]
                for u in range(8):
                    prev = ec[0]
                    svec, xvec, nvec = ec[1:9], ec[9:17], ec[17:25]
                    d = dvec[u]
                    bdy = d != prev

                    @pl.when(bdy)
                    def _():
                        flush(prev, svec, xvec, nvec)

                    ro = (gi * 8 + u) * 128
                    rows = [rowf[pl.ds(ro + 16 * k, 16)] for k in range(8)]
                    m = jnp.where(bdy, 1.0, 0.0)
                    kv = jnp.broadcast_to(1.0 - m, (16,))
                    negv = jnp.broadcast_to(m * _NEG, (16,))
                    posv = jnp.broadcast_to(m * _POS, (16,))
                    sn = tuple(svec[k] * kv + rows[k] for k in range(8))
                    xn = tuple(
                        jnp.maximum(xvec[k] * kv + negv, rows[k])
                        for k in range(8))
                    nn = tuple(
                        jnp.minimum(nvec[k] * kv + posv, rows[k])
                        for k in range(8))
                    ec = (d,) + sn + xn + nn
                return ec

            return lax.fori_loop(0, _C2 // 8, group_body, carry)

        z = jnp.zeros((16,), jnp.float32)
        init = ((jnp.int32(-1),) + (z,) * 8
                + (jnp.full((16,), _NEG, jnp.float32),) * 8
                + (jnp.full((16,), _POS, jnp.float32),) * 8)
        fin = lax.fori_loop(0, nch, chunk_body, init)
        flush(fin[0], fin[1:9], fin[9:17], fin[17:25])
        pltpu.sync_copy(acc_s, osum_hbm.at[pl.ds(node_lo * 128, _RN * 128)])
        pltpu.sync_copy(acc_x, omax_hbm.at[pl.ds(node_lo * 128, _RN * 128)])
        pltpu.sync_copy(acc_n, omin_hbm.at[pl.ds(node_lo * 128, _RN * 128)])


@functools.cache
def _sc_segred():
    return pl.kernel(
        _sc_segred_body,
        out_type=[jax.ShapeDtypeStruct((_NP * 128,), jnp.float32)] * 3,
        mesh=plsc.VectorSubcoreMesh(core_axis_name="c", subcore_axis_name="s"),
        scratch_types=(
            [pltpu.VMEM((8 * _NW + 16,), jnp.int32)]
            + [pltpu.VMEM((_C2 + 16,), jnp.int32)]
            + [pltpu.VMEM((_C2 * 128,), jnp.float32)]
            + [pltpu.VMEM((_RN * 128,), jnp.float32)] * 3
        ),
    )


def _segment_reduce(e_new_pad, sdst_p, bnd):
    enf = e_new_pad.reshape(-1)
    osum, omax, omin = _sc_segred()(enf, sdst_p, bnd)
    return (osum.reshape(_NP, 128), omax.reshape(_NP, 128),
            omin.reshape(_NP, 128))


def kernel(x, e, g, edges, node_idx, edge_idx, steps, params):
    f32 = jnp.float32
    src, dst = edges[0], edges[1]

    def row(v):
        return v.reshape(1, -1).astype(f32)

    w_ence, b_ence = params['enc_e']
    w_encx, b_encx = params['enc_x']
    w_encg, b_encg = params['enc_g']
    w_ce, b_ce = params['core_e']
    we0, we1 = w_ce[0:128], w_ce[128:256]
    ws0, ws1 = w_ce[256:384], w_ce[384:512]
    wd0, wd1 = w_ce[512:640], w_ce[640:768]
    wg2 = w_ce[768:770]
    w_an, b_an = params['agg_n']
    a1, a2, a3, a4 = w_an[0:128], w_an[128:256], w_an[256:384], w_an[384:512]
    w_cx, b_cx = params['core_x']
    x0w, x1w, xaw, wgx = (w_cx[0:128], w_cx[128:256], w_cx[256:384],
                          w_cx[384:386])
    w_ge, b_ge = params['agg_ge']
    w_gn, b_gn = params['agg_gn']
    w_cg, b_cg = params['core_g']
    wcg = jnp.zeros((4, 128), f32)
    wcg = wcg.at[0, 0].set(w_cg[0, 0]).at[0, 1].set(w_cg[1, 0])
    wcg = wcg.at[0, 2].set(b_cg[0])
    wcg = wcg.at[1, :].set(w_cg[2:130, 0]).at[2, :].set(w_cg[130:258, 0])
    w_de1, b_de1 = params['dec_e1']
    w_de2, b_de2 = params['dec_e2']
    w_dx1, b_dx1 = params['dec_x1']
    w_dx2, b_dx2 = params['dec_x2']
    w_dg, b_dg = params['dec_g']
    w_oe, b_oe = params['out_e']
    w_ox, b_ox = params['out_x']
    w_og, b_og = params['out_g']
    wgp = jnp.zeros((16, 128), f32).at[:, 0:1].set(w_encg)
    bgp = jnp.zeros((1, 128), f32).at[0, 0].set(b_encg[0])
    wdgp = jnp.zeros((1, 128), f32)
    wdgp = wdgp.at[0, 0].set(w_dg[0, 0]).at[0, 1].set(b_dg[0])
    wdgp = wdgp.at[0, 2].set(w_og[0, 0]).at[0, 3].set(b_og[0])

    sidx = jnp.argsort(dst).astype(jnp.int32)
    sdst = dst[sidx]
    src_s = src[sidx]
    inv = jnp.zeros((E,), jnp.int32).at[sidx].set(
        jnp.arange(E, dtype=jnp.int32))
    rowptr = jnp.searchsorted(sdst, jnp.arange(N + 1, dtype=jnp.int32)
                              ).astype(jnp.int32)
    cnt = (rowptr[1:] - rowptr[:-1]).astype(f32).reshape(N, 1)
    rbound = rowptr[jnp.minimum(jnp.arange(_NR + 1) * _RN, N)]
    rlo = jnp.bitwise_and(rbound[:_NR], jnp.int32(~7))
    rhi = rbound[1:_NR + 1]
    bnd = (jnp.zeros((_NW, 8), jnp.int32)
           .at[:, 0].set(rlo[0::2]).at[:, 1].set(rhi[0::2])
           .at[:, 2].set(rlo[1::2]).at[:, 3].set(rhi[1::2])
           .reshape(-1))
    sdst_p = jnp.concatenate([sdst, jnp.full((_C2,), N, jnp.int32)])

    e0u, p0u = _enc_edges(e, w_ence, row(b_ence), we0, row(b_ce))
    e0 = _sc_permute()(e0u, sidx)
    p0 = _sc_permute()(p0u, sidx)
    x0, xs0, xd0, xc0 = _enc_nodes(x, w_encx, row(b_encx), ws0, wd0, x0w,
                                   row(b_cx))
    genc = _g_enc(g, wgp, bgp)
    g0b = jnp.broadcast_to(genc[:, 0:1], (1, 128))

    def body(_, carry):
        ecur, xcur, gb = carry
        xs, xd = _node_prep(xcur, ws1, wd1, xs0, xd0)
        xg = _gather_xg(xs, xd, src_s, sdst)
        e_new, esum, emax, emin = _edge_core(ecur, p0, xg, g0b, gb, wg2, we1)
        nsum, nmax, nmin = _segment_reduce(e_new, sdst_p, bnd)
        x_new, xsum, xmax, xmin = _node_core(
            xcur, xc0, nsum, nmax, nmin, cnt, a1, a2, a3, a4, row(b_an),
            x1w, xaw, g0b, gb, wgx)
        gb_new = _global_core(esum, emax, emin, xsum, xmax, xmin, g0b, gb,
                              w_ge, row(b_ge), w_gn, row(b_gn), wcg)
        return (e_new, x_new, gb_new)

    ef, xf, gbf = jax.lax.fori_loop(0, steps, body, (e0, x0, g0b))

    def decode(_):
        ef_orig = _sc_permute()(ef, inv)
        oe = _decode(ef_orig, w_de1, row(b_de1), w_de2, row(b_de2),
                     row(w_oe[:, 0]), row(jnp.broadcast_to(b_oe, (128,))),
                     E, BE)
        ox = _decode(xf, w_dx1, row(b_dx1), w_dx2, row(b_dx2),
                     row(w_ox[:, 0]), row(jnp.broadcast_to(b_ox, (128,))),
                     N, BN)
        og = _dec_g(gbf, wdgp)[:, 0:1]
        return ox, oe, og

    def zeros(_):
        return (jnp.zeros((N, 1), f32), jnp.zeros((E, 1), f32),
                jnp.zeros((1, 1), f32))

    return jax.lax.cond(steps > 0, decode, zeros, None)


# speedup vs baseline: 1.1492x; 1.1492x over previous
import functools

import jax
import jax.numpy as jnp
from jax import lax
from jax.experimental import pallas as pl
from jax.experimental.pallas import tpu as pltpu
from jax.experimental.pallas import tpu_sc as plsc

INTERPRET = False

N = 10000
E = 320000
BE = 1600
BN = 1000


def _leaky(h):
    return jnp.where(h >= 0, h, 0.01 * h)


def _ln(h):
    mu = jnp.mean(h, axis=-1, keepdims=True)
    var = jnp.mean((h - mu) ** 2, axis=-1, keepdims=True)
    return (h - mu) * jax.lax.rsqrt(var + 1e-5)


def _vspec(shape):
    return pl.BlockSpec(shape, lambda i: (0,) * len(shape))


def _enc_edges_body(e_in, wenc, benc, we0, bcore, e_enc, p0):
    ee = _leaky(jnp.dot(e_in[...], wenc[...]) + benc[...])
    e_enc[...] = ee
    p0[...] = jnp.dot(ee, we0[...]) + bcore[...]


def _enc_edges(e_in, wenc, benc, we0, bcore):
    g = E // BE
    return pl.pallas_call(
        _enc_edges_body,
        grid=(g,),
        in_specs=[
            pl.BlockSpec((BE, 16), lambda i: (i, 0)),
            _vspec((16, 128)),
            _vspec((1, 128)),
            _vspec((128, 128)),
            _vspec((1, 128)),
        ],
        out_specs=[
            pl.BlockSpec((BE, 128), lambda i: (i, 0)),
            pl.BlockSpec((BE, 128), lambda i: (i, 0)),
        ],
        out_shape=[
            jax.ShapeDtypeStruct((E + 128, 128), jnp.float32),
            jax.ShapeDtypeStruct((E + 128, 128), jnp.float32),
        ],
        interpret=INTERPRET,
    )(e_in, wenc, benc, we0, bcore)


def _enc_nodes_body(x_in, wenc, benc, ws0, wd0, wx0, bx, x_enc, xs0, xd0, xc0):
    xe = _leaky(jnp.dot(x_in[...], wenc[...]) + benc[...])
    x_enc[...] = xe
    xs0[...] = jnp.dot(xe, ws0[...])
    xd0[...] = jnp.dot(xe, wd0[...])
    xc0[...] = jnp.dot(xe, wx0[...]) + bx[...]


def _enc_nodes(x_in, wenc, benc, ws0, wd0, wx0, bx):
    g = N // BN
    return pl.pallas_call(
        _enc_nodes_body,
        grid=(g,),
        in_specs=[
            pl.BlockSpec((BN, 128), lambda i: (i, 0)),
            _vspec((128, 128)),
            _vspec((1, 128)),
            _vspec((128, 128)),
            _vspec((128, 128)),
            _vspec((128, 128)),
            _vspec((1, 128)),
        ],
        out_specs=[pl.BlockSpec((BN, 128), lambda i: (i, 0))] * 4,
        out_shape=[jax.ShapeDtypeStruct((N, 128), jnp.float32)] * 4,
        interpret=INTERPRET,
    )(x_in, wenc, benc, ws0, wd0, wx0, bx)


def _g_enc_body(g_in, w, b, out):
    out[...] = _leaky(jnp.dot(g_in[...], w[...]) + b[...])


def _g_enc(g_in, wpad, bpad):
    return pl.pallas_call(
        _g_enc_body,
        grid=(1,),
        in_specs=[_vspec((1, 16)), _vspec((16, 128)), _vspec((1, 128))],
        out_specs=_vspec((1, 128)),
        out_shape=jax.ShapeDtypeStruct((1, 128), jnp.float32),
        interpret=INTERPRET,
    )(g_in, wpad, bpad)


def _node_prep_body(x, ws1, wd1, xs0, xd0, xs, xd):
    xv = x[...]
    xs[...] = jnp.dot(xv, ws1[...]) + xs0[...]
    xd[...] = jnp.dot(xv, wd1[...]) + xd0[...]


def _node_prep(x, ws1, wd1, xs0, xd0):
    g = N // BN
    return pl.pallas_call(
        _node_prep_body,
        grid=(g,),
        in_specs=[
            pl.BlockSpec((BN, 128), lambda i: (i, 0)),
            _vspec((128, 128)),
            _vspec((128, 128)),
            pl.BlockSpec((BN, 128), lambda i: (i, 0)),
            pl.BlockSpec((BN, 128), lambda i: (i, 0)),
        ],
        out_specs=[pl.BlockSpec((BN, 128), lambda i: (i, 0))] * 2,
        out_shape=[jax.ShapeDtypeStruct((N, 128), jnp.float32)] * 2,
        interpret=INTERPRET,
    )(x, ws1, wd1, xs0, xd0)


def _edge_core_body(e, p0, xg, g0b, gb, wg2, we1, e_new, esum, emax, emin):
    i = pl.program_id(0)
    grow = g0b[...] * wg2[0:1, :] + gb[...] * wg2[1:2, :]
    h = jnp.dot(e[...], we1[...]) + p0[...] + xg[...] + grow
    y = _ln(_leaky(h))
    e_new[...] = y
    bs = jnp.sum(y, axis=0, keepdims=True)
    bmx = jnp.max(y, axis=0, keepdims=True)
    bmn = jnp.min(y, axis=0, keepdims=True)

    @pl.when(i == 0)
    def _():
        esum[...] = bs
        emax[...] = bmx
        emin[...] = bmn

    @pl.when(i != 0)
    def _():
        esum[...] += bs
        emax[...] = jnp.maximum(emax[...], bmx)
        emin[...] = jnp.minimum(emin[...], bmn)


def _edge_core(e, p0, xg, g0b, gb, wg2, we1):
    g = E // BE
    return pl.pallas_call(
        _edge_core_body,
        grid=(g,),
        in_specs=[
            pl.BlockSpec((BE, 128), lambda i: (i, 0)),
            pl.BlockSpec((BE, 128), lambda i: (i, 0)),
            pl.BlockSpec((BE, 128), lambda i: (i, 0)),
            _vspec((1, 128)),
            _vspec((1, 128)),
            _vspec((2, 128)),
            _vspec((128, 128)),
        ],
        out_specs=[
            pl.BlockSpec((BE, 128), lambda i: (i, 0)),
            _vspec((1, 128)),
            _vspec((1, 128)),
            _vspec((1, 128)),
        ],
        out_shape=[
            jax.ShapeDtypeStruct((E + 128, 128), jnp.float32),
            jax.ShapeDtypeStruct((1, 128), jnp.float32),
            jax.ShapeDtypeStruct((1, 128), jnp.float32),
            jax.ShapeDtypeStruct((1, 128), jnp.float32),
        ],
        interpret=INTERPRET,
    )(e, p0, xg, g0b, gb, wg2, we1)


def _node_core_body(x, xc0, nsum, nmax, nmin, cnt, a1, a2, a3, a4, bagg, x1w,
                    xaw, g0b, gb, wgx, x_new, xsum, xmax, xmin):
    i = pl.program_id(0)
    c = cnt[...]
    has = c > 0.0
    mx = jnp.where(has, nmax[...], 0.0)
    mn = jnp.where(has, nmin[...], 0.0)
    s = jnp.where(has, nsum[...], 0.0)
    mean = s / jnp.maximum(c, 1.0)
    agg = _leaky(
        jnp.dot(s, a1[...]) + jnp.dot(mx, a2[...]) + jnp.dot(mean, a3[...])
        + jnp.dot(mn, a4[...]) + bagg[...]
    )
    grow = g0b[...] * wgx[0:1, :] + gb[...] * wgx[1:2, :]
    h = jnp.dot(x[...], x1w[...]) + xc0[...] + jnp.dot(agg, xaw[...]) + grow
    y = _ln(_leaky(h))
    x_new[...] = y
    bs = jnp.sum(y, axis=0, keepdims=True)
    bmx = jnp.max(y, axis=0, keepdims=True)
    bmn = jnp.min(y, axis=0, keepdims=True)

    @pl.when(i == 0)
    def _():
        xsum[...] = bs
        xmax[...] = bmx
        xmin[...] = bmn

    @pl.when(i != 0)
    def _():
        xsum[...] += bs
        xmax[...] = jnp.maximum(xmax[...], bmx)
        xmin[...] = jnp.minimum(xmin[...], bmn)


def _node_core(x, xc0, nsum, nmax, nmin, cnt, a1, a2, a3, a4, bagg, x1w, xaw,
               g0b, gb, wgx):
    g = N // BN
    bspec = pl.BlockSpec((BN, 128), lambda i: (i, 0))
    return pl.pallas_call(
        _node_core_body,
        grid=(g,),
        in_specs=[
            bspec, bspec, bspec, bspec, bspec,
            pl.BlockSpec((BN, 1), lambda i: (i, 0)),
            _vspec((128, 128)), _vspec((128, 128)), _vspec((128, 128)),
            _vspec((128, 128)), _vspec((1, 128)),
            _vspec((128, 128)), _vspec((128, 128)),
            _vspec((1, 128)), _vspec((1, 128)), _vspec((2, 128)),
        ],
        out_specs=[
            bspec,
            _vspec((1, 128)), _vspec((1, 128)), _vspec((1, 128)),
        ],
        out_shape=[
            jax.ShapeDtypeStruct((N, 128), jnp.float32),
            jax.ShapeDtypeStruct((1, 128), jnp.float32),
            jax.ShapeDtypeStruct((1, 128), jnp.float32),
            jax.ShapeDtypeStruct((1, 128), jnp.float32),
        ],
        interpret=INTERPRET,
    )(x, xc0, nsum, nmax, nmin, cnt, a1, a2, a3, a4, bagg, x1w, xaw, g0b, gb,
      wgx)


def _global_body(esum, emax, emin, xsum, xmax, xmin, g0b, gb, ge, bge, gn, bgn,
                 wcg, gb_new):
    es = esum[...]
    eagg = _leaky(
        jnp.dot(es, ge[0:128, :]) + jnp.dot(emax[...], ge[128:256, :])
        + jnp.dot(es * (1.0 / E), ge[256:384, :])
        + jnp.dot(emin[...], ge[384:512, :]) + bge[...]
    )
    xs = xsum[...]
    nagg = _leaky(
        jnp.dot(xs, gn[0:128, :]) + jnp.dot(xmax[...], gn[128:256, :])
        + jnp.dot(xs * (1.0 / N), gn[256:384, :])
        + jnp.dot(xmin[...], gn[384:512, :]) + bgn[...]
    )
    h = (
        g0b[0:1, 0:1] * wcg[0:1, 0:1] + gb[0:1, 0:1] * wcg[0:1, 1:2]
        + wcg[0:1, 2:3]
        + jnp.sum(eagg * wcg[1:2, :], axis=-1, keepdims=True)
        + jnp.sum(nagg * wcg[2:3, :], axis=-1, keepdims=True)
    )
    y = _leaky(h)
    gb_new[...] = jnp.broadcast_to((y - y) * jax.lax.rsqrt(1e-5),
                                   gb_new.shape)


def _global_core(esum, emax, emin, xsum, xmax, xmin, g0b, gb, ge, bge, gn, bgn,
                 wcg):
    return pl.pallas_call(
        _global_body,
        grid=(1,),
        in_specs=[
            _vspec((1, 128)), _vspec((1, 128)), _vspec((1, 128)),
            _vspec((1, 128)), _vspec((1, 128)), _vspec((1, 128)),
            _vspec((1, 128)), _vspec((1, 128)),
            _vspec((512, 128)), _vspec((1, 128)),
            _vspec((512, 128)), _vspec((1, 128)),
            _vspec((4, 128)),
        ],
        out_specs=_vspec((1, 128)),
        out_shape=jax.ShapeDtypeStruct((1, 128), jnp.float32),
        interpret=INTERPRET,
    )(esum, emax, emin, xsum, xmax, xmin, g0b, gb, ge, bge, gn, bgn, wcg)


def _dec_body(ow, z, d1, b1, d2, b2, wout, bout, out):
    h = _leaky(jnp.dot(z[...], d1[...]) + b1[...])
    h = _leaky(jnp.dot(h, d2[...]) + b2[...])
    val = jnp.sum(h * wout[...], axis=-1, keepdims=True) + bout[0:1, 0:1]
    out[...] = jnp.broadcast_to(val, out.shape)


def _decode(z, d1, b1, d2, b2, wout, bout, total, blk, ow=1):
    g = total // blk
    return pl.pallas_call(
        functools.partial(_dec_body, ow),
        grid=(g,),
        in_specs=[
            pl.BlockSpec((blk, 128), lambda i: (i, 0)),
            _vspec((128, 128)), _vspec((1, 128)),
            _vspec((128, 128)), _vspec((1, 128)),
            _vspec((1, 128)), _vspec((1, 128)),
        ],
        out_specs=pl.BlockSpec((blk, ow), lambda i: (i, 0)),
        out_shape=jax.ShapeDtypeStruct((total, ow), jnp.float32),
        interpret=INTERPRET,
    )(z, d1, b1, d2, b2, wout, bout)


def _dec_g_body(gb, wpack, out):
    h = _leaky(gb[...] * wpack[0:1, 0:1] + wpack[0:1, 1:2])
    out[...] = h * wpack[0:1, 2:3] + wpack[0:1, 3:4]


def _dec_g(gb, wpack):
    return pl.pallas_call(
        _dec_g_body,
        grid=(1,),
        in_specs=[_vspec((1, 128)), _vspec((1, 128))],
        out_specs=_vspec((1, 128)),
        out_shape=jax.ShapeDtypeStruct((1, 128), jnp.float32),
        interpret=INTERPRET,
    )(gb, wpack)


_NW = 32
_SPAN = E // _NW
_C = 80
_NCH = _SPAN // _C
_NB = 5


def _sc_gather_body(xs_hbm, xd_hbm, src_hbm, dst_hbm, out_hbm, src_v, dst_v,
                    *rest):
    bufs = rest[0:_NB]
    g1s = rest[_NB:2 * _NB]
    g2s = rest[2 * _NB:3 * _NB]
    wbs = rest[3 * _NB:4 * _NB]
    wid = lax.axis_index("s") * 2 + lax.axis_index("c")
    base = wid * _SPAN
    pltpu.sync_copy(src_hbm.at[pl.ds(base, _SPAN)], src_v)
    pltpu.sync_copy(dst_hbm.at[pl.ds(base, _SPAN)], dst_v)

    def g1_start(c, b):
        pltpu.async_copy(xs_hbm.at[src_v.at[pl.ds(c * _C, _C)]], bufs[b],
                         g1s[b])

    def g1_wait(b):
        pltpu.make_async_copy(xs_hbm.at[src_v.at[pl.ds(0, _C)]], bufs[b],
                              g1s[b]).wait()

    for b in range(_NB):
        g1_start(b, b)

    def round_body(it, _):
        c0 = it * _NB
        for b in range(_NB):
            g1_wait(b)
            pltpu.async_copy(xd_hbm.at[dst_v.at[pl.ds((c0 + b) * _C, _C)]],
                             bufs[b], g2s[b], add=True)
        for b in range(_NB):
            pltpu.make_async_copy(xd_hbm.at[dst_v.at[pl.ds(0, _C)]], bufs[b],
                                  g2s[b]).wait()
            pltpu.async_copy(
                bufs[b], out_hbm.at[pl.ds(base + (c0 + b) * _C, _C), :],
                wbs[b])
        for b in range(_NB):
            nxt = c0 + _NB + b

            @pl.when(nxt < _NCH)
            def _():
                pltpu.make_async_copy(
                    bufs[b], out_hbm.at[pl.ds(base, _C), :], wbs[b]).wait()
                g1_start(nxt, b)
        return 0

    lax.fori_loop(0, _NCH // _NB, round_body, 0)
    for b in range(_NB):
        pltpu.make_async_copy(bufs[b], out_hbm.at[pl.ds(base, _C), :],
                              wbs[b]).wait()


@functools.cache
def _sc_gather():
    return pl.kernel(
        _sc_gather_body,
        out_type=jax.ShapeDtypeStruct((E, 128), jnp.float32),
        mesh=plsc.VectorSubcoreMesh(core_axis_name="c", subcore_axis_name="s"),
        scratch_types=(
            [pltpu.VMEM((_SPAN,), jnp.int32)] * 2
            + [pltpu.VMEM((_C, 128), jnp.float32)] * _NB
            + [pltpu.SemaphoreType.DMA] * (3 * _NB)
        ),
    )


def _gather_xg(xs, xd, src, dst):
    return _sc_gather()(xs, xd, src, dst)


def _sc_permute_body(C, D, table_hbm, idx_hbm, out_hbm, idx_v, *rest):
    nch = _SPAN // C
    nb = 5
    bufs = rest[0:nb]
    g1s = rest[nb:2 * nb]
    wbs = rest[2 * nb:3 * nb]
    wid = lax.axis_index("s") * 2 + lax.axis_index("c")
    base = wid * _SPAN
    pltpu.sync_copy(idx_hbm.at[pl.ds(base, _SPAN)], idx_v)

    def g1_start(c, b):
        pltpu.async_copy(table_hbm.at[idx_v.at[pl.ds(c * C, C)]], bufs[b],
                         g1s[b])

    for b in range(nb):
        g1_start(b, b)

    def round_body(it, _):
        c0 = it * nb
        for b in range(nb):
            pltpu.make_async_copy(table_hbm.at[idx_v.at[pl.ds(0, C)]],
                                  bufs[b], g1s[b]).wait()
            pltpu.async_copy(bufs[b],
                             out_hbm.at[pl.ds(base + (c0 + b) * C, C), :],
                             wbs[b])
        for b in range(nb):
            nxt = c0 + nb + b

            @pl.when(nxt < nch)
            def _():
                pltpu.make_async_copy(bufs[b], out_hbm.at[pl.ds(base, C), :],
                                      wbs[b]).wait()
                g1_start(nxt, b)
        return 0

    lax.fori_loop(0, nch // nb, round_body, 0)
    for b in range(nb):
        pltpu.make_async_copy(bufs[b], out_hbm.at[pl.ds(base, C), :],
                              wbs[b]).wait()


@functools.cache
def _sc_permute():
    C = 80
    nb = 5
    D = 128
    return pl.kernel(
        functools.partial(_sc_permute_body, C, D),
        out_type=jax.ShapeDtypeStruct((E + 128, D), jnp.float32),
        mesh=plsc.VectorSubcoreMesh(core_axis_name="c", subcore_axis_name="s"),
        scratch_types=(
            [pltpu.VMEM((_SPAN,), jnp.int32)]
            + [pltpu.VMEM((C, D), jnp.float32)] * nb
            + [pltpu.SemaphoreType.DMA] * (2 * nb)
        ),
    )


_RN = 160
_NR = 64
_NP = _NR * _RN
_C2 = 128
_NEG = -3.0e38
_POS = 3.0e38


def _sc_segred_body(enf_hbm, sdst_hbm, bnd_hbm,
                    osum_hbm, omax_hbm, omin_hbm,
                    bnd_v, sdst_v, rowf, acc_s, acc_x, acc_n):
    wid = lax.axis_index("s") * 2 + lax.axis_index("c")
    pltpu.sync_copy(bnd_hbm, bnd_v.at[pl.ds(0, 8 * _NW)])
    bvec = bnd_v[pl.ds(wid * 8, 16)]
    for j in range(2):
        r = wid * 2 + j
        lo_e = pl.multiple_of(bvec[2 * j] & ~7, 8)
        hi_e = bvec[2 * j + 1]
        node_lo = r * _RN
        node_hi = jnp.minimum(node_lo + _RN, N)
        nch = lax.shift_right_logical(hi_e - lo_e + (_C2 - 1), 7)

        def flush(prev, svec, xvec, nvec):
            in_r = jnp.logical_and(prev >= node_lo, prev < node_hi)

            @pl.when(in_r)
            def _():
                off = (prev - node_lo) * 128
                for k in range(8):
                    acc_s[pl.ds(off + 16 * k, 16)] = svec[k]
                    acc_x[pl.ds(off + 16 * k, 16)] = xvec[k]
                    acc_n[pl.ds(off + 16 * k, 16)] = nvec[k]

        def chunk_body(ci, carry):
            pos = pl.multiple_of(lo_e + ci * _C2, 8)
            pltpu.sync_copy(sdst_hbm.at[pl.ds(pos, _C2)],
                            sdst_v.at[pl.ds(0, _C2)])
            pltpu.sync_copy(enf_hbm.at[pl.ds(pos * 128, _C2 * 128)], rowf)

            def group_body(gi, ec):
                dvec = sdst_v[pl.ds(gi * 8, 16)]
                for u in range(8):
                    prev = ec[0]
                    svec, xvec, nvec = ec[1:9], ec[9:17], ec[17:25]
                    d = dvec[u]
                    bdy = d != prev

                    @pl.when(bdy)
                    def _():
                        flush(prev, svec, xvec, nvec)

                    ro = (gi * 8 + u) * 128
                    rows = [rowf[pl.ds(ro + 16 * k, 16)] for k in range(8)]
                    m = jnp.where(bdy, 1.0, 0.0)
                    kv = jnp.broadcast_to(1.0 - m, (16,))
                    negv = jnp.broadcast_to(m * _NEG, (16,))
                    posv = jnp.broadcast_to(m * _POS, (16,))
                    sn = tuple(svec[k] * kv + rows[k] for k in range(8))
                    xn = tuple(
                        jnp.maximum(xvec[k] * kv + negv, rows[k])
                        for k in range(8))
                    nn = tuple(
                        jnp.minimum(nvec[k] * kv + posv, rows[k])
                        for k in range(8))
                    ec = (d,) + sn + xn + nn
                return ec

            return lax.fori_loop(0, _C2 // 8, group_body, carry)

        z = jnp.zeros((16,), jnp.float32)
        init = ((jnp.int32(-1),) + (z,) * 8
                + (jnp.full((16,), _NEG, jnp.float32),) * 8
                + (jnp.full((16,), _POS, jnp.float32),) * 8)
        fin = lax.fori_loop(0, nch, chunk_body, init)
        flush(fin[0], fin[1:9], fin[9:17], fin[17:25])
        pltpu.sync_copy(acc_s, osum_hbm.at[pl.ds(node_lo * 128, _RN * 128)])
        pltpu.sync_copy(acc_x, omax_hbm.at[pl.ds(node_lo * 128, _RN * 128)])
        pltpu.sync_copy(acc_n, omin_hbm.at[pl.ds(node_lo * 128, _RN * 128)])


@functools.cache
def _sc_segred():
    return pl.kernel(
        _sc_segred_body,
        out_type=[jax.ShapeDtypeStruct((_NP * 128,), jnp.float32)] * 3,
        mesh=plsc.VectorSubcoreMesh(core_axis_name="c", subcore_axis_name="s"),
        scratch_types=(
            [pltpu.VMEM((8 * _NW + 16,), jnp.int32)]
            + [pltpu.VMEM((_C2 + 16,), jnp.int32)]
            + [pltpu.VMEM((_C2 * 128,), jnp.float32)]
            + [pltpu.VMEM((_RN * 128,), jnp.float32)] * 3
        ),
    )


def _segment_reduce(e_new_pad, sdst_p, bnd):
    enf = e_new_pad.reshape(-1)
    osum, omax, omin = _sc_segred()(enf, sdst_p, bnd)
    return (osum.reshape(_NP, 128), omax.reshape(_NP, 128),
            omin.reshape(_NP, 128))


def kernel(x, e, g, edges, node_idx, edge_idx, steps, params):
    f32 = jnp.float32
    src, dst = edges[0], edges[1]

    def row(v):
        return v.reshape(1, -1).astype(f32)

    w_ence, b_ence = params['enc_e']
    w_encx, b_encx = params['enc_x']
    w_encg, b_encg = params['enc_g']
    w_ce, b_ce = params['core_e']
    we0, we1 = w_ce[0:128], w_ce[128:256]
    ws0, ws1 = w_ce[256:384], w_ce[384:512]
    wd0, wd1 = w_ce[512:640], w_ce[640:768]
    wg2 = w_ce[768:770]
    w_an, b_an = params['agg_n']
    a1, a2, a3, a4 = w_an[0:128], w_an[128:256], w_an[256:384], w_an[384:512]
    w_cx, b_cx = params['core_x']
    x0w, x1w, xaw, wgx = (w_cx[0:128], w_cx[128:256], w_cx[256:384],
                          w_cx[384:386])
    w_ge, b_ge = params['agg_ge']
    w_gn, b_gn = params['agg_gn']
    w_cg, b_cg = params['core_g']
    wcg = jnp.zeros((4, 128), f32)
    wcg = wcg.at[0, 0].set(w_cg[0, 0]).at[0, 1].set(w_cg[1, 0])
    wcg = wcg.at[0, 2].set(b_cg[0])
    wcg = wcg.at[1, :].set(w_cg[2:130, 0]).at[2, :].set(w_cg[130:258, 0])
    w_de1, b_de1 = params['dec_e1']
    w_de2, b_de2 = params['dec_e2']
    w_dx1, b_dx1 = params['dec_x1']
    w_dx2, b_dx2 = params['dec_x2']
    w_dg, b_dg = params['dec_g']
    w_oe, b_oe = params['out_e']
    w_ox, b_ox = params['out_x']
    w_og, b_og = params['out_g']
    wgp = jnp.zeros((16, 128), f32).at[:, 0:1].set(w_encg)
    bgp = jnp.zeros((1, 128), f32).at[0, 0].set(b_encg[0])
    wdgp = jnp.zeros((1, 128), f32)
    wdgp = wdgp.at[0, 0].set(w_dg[0, 0]).at[0, 1].set(b_dg[0])
    wdgp = wdgp.at[0, 2].set(w_og[0, 0]).at[0, 3].set(b_og[0])

    dst_p = jnp.concatenate([dst, jnp.full((128,), N, jnp.int32)])
    cnt = jax.ops.segment_sum(jnp.ones((E, 1), f32), dst,
                              num_segments=N)

    e0, p0 = _enc_edges(e, w_ence, row(b_ence), we0, row(b_ce))
    x0, xs0, xd0, xc0 = _enc_nodes(x, w_encx, row(b_encx), ws0, wd0, x0w,
                                   row(b_cx))
    genc = _g_enc(g, wgp, bgp)
    g0b = jnp.broadcast_to(genc[:, 0:1], (1, 128))

    def body(_, carry):
        ecur, xcur, gb = carry
        xs, xd = _node_prep(xcur, ws1, wd1, xs0, xd0)
        xg = _gather_xg(xs, xd, src, dst)
        e_new, esum, emax, emin = _edge_core(ecur, p0, xg, g0b, gb, wg2, we1)
        nsum = jax.ops.segment_sum(e_new, dst_p, num_segments=N + 1)[:N]
        nmax = jax.ops.segment_max(e_new, dst_p, num_segments=N + 1)[:N]
        nmin = -jax.ops.segment_max(-e_new, dst_p, num_segments=N + 1)[:N]
        x_new, xsum, xmax, xmin = _node_core(
            xcur, xc0, nsum, nmax, nmin, cnt, a1, a2, a3, a4, row(b_an),
            x1w, xaw, g0b, gb, wgx)
        gb_new = _global_core(esum, emax, emin, xsum, xmax, xmin, g0b, gb,
                              w_ge, row(b_ge), w_gn, row(b_gn), wcg)
        return (e_new, x_new, gb_new)

    ef, xf, gbf = jax.lax.fori_loop(0, steps, body, (e0, x0, g0b))

    def decode(_):
        oe = _decode(ef, w_de1, row(b_de1), w_de2, row(b_de2),
                     row(w_oe[:, 0]), row(jnp.broadcast_to(b_oe, (128,))),
                     E, BE)
        ox = _decode(xf, w_dx1, row(b_dx1), w_dx2, row(b_dx2),
                     row(w_ox[:, 0]), row(jnp.broadcast_to(b_ox, (128,))),
                     N, BN)
        og = _dec_g(gbf, wdgp)[:, 0:1]
        return ox, oe, og

    def zeros(_):
        return (jnp.zeros((N, 1), f32), jnp.zeros((E, 1), f32),
                jnp.zeros((1, 1), f32))

    return jax.lax.cond(steps > 0, decode, zeros, None)


# speedup vs baseline: 1.1494x; 1.0002x over previous
import functools

import jax
import jax.numpy as jnp
from jax import lax
from jax.experimental import pallas as pl
from jax.experimental.pallas import tpu as pltpu
from jax.experimental.pallas import tpu_sc as plsc

INTERPRET = False

N = 10000
E = 320000
BE = 1600
BN = 1000


def _dot(a, b):
    return jnp.dot(a, b)

def _leaky(h):
    return jnp.where(h >= 0, h, 0.01 * h)


def _ln(h):
    mu = jnp.mean(h, axis=-1, keepdims=True)
    var = jnp.mean((h - mu) ** 2, axis=-1, keepdims=True)
    return (h - mu) * jax.lax.rsqrt(var + 1e-5)


def _vspec(shape):
    return pl.BlockSpec(shape, lambda i: (0,) * len(shape))


def _enc_edges_body(e_in, wenc, benc, we0, bcore, e_enc, p0):
    ee = _leaky(_dot(e_in[...], wenc[...]) + benc[...])
    e_enc[...] = ee
    p0[...] = _dot(ee, we0[...]) + bcore[...]


def _enc_edges(e_in, wenc, benc, we0, bcore):
    g = E // BE
    return pl.pallas_call(
        _enc_edges_body,
        grid=(g,),
        in_specs=[
            pl.BlockSpec((BE, 16), lambda i: (i, 0)),
            _vspec((16, 128)),
            _vspec((1, 128)),
            _vspec((128, 128)),
            _vspec((1, 128)),
        ],
        out_specs=[
            pl.BlockSpec((BE, 128), lambda i: (i, 0)),
            pl.BlockSpec((BE, 128), lambda i: (i, 0)),
        ],
        out_shape=[
            jax.ShapeDtypeStruct((E + 128, 128), jnp.float32),
            jax.ShapeDtypeStruct((E + 128, 128), jnp.float32),
        ],
        interpret=INTERPRET,
    )(e_in, wenc, benc, we0, bcore)


def _enc_nodes_body(x_in, wenc, benc, ws0, wd0, wx0, bx, x_enc, xs0, xd0, xc0):
    xe = _leaky(_dot(x_in[...], wenc[...]) + benc[...])
    x_enc[...] = xe
    xs0[...] = _dot(xe, ws0[...])
    xd0[...] = _dot(xe, wd0[...])
    xc0[...] = _dot(xe, wx0[...]) + bx[...]


def _enc_nodes(x_in, wenc, benc, ws0, wd0, wx0, bx):
    g = N // BN
    return pl.pallas_call(
        _enc_nodes_body,
        grid=(g,),
        in_specs=[
            pl.BlockSpec((BN, 128), lambda i: (i, 0)),
            _vspec((128, 128)),
            _vspec((1, 128)),
            _vspec((128, 128)),
            _vspec((128, 128)),
            _vspec((128, 128)),
            _vspec((1, 128)),
        ],
        out_specs=[pl.BlockSpec((BN, 128), lambda i: (i, 0))] * 4,
        out_shape=[jax.ShapeDtypeStruct((N, 128), jnp.float32)] * 4,
        interpret=INTERPRET,
    )(x_in, wenc, benc, ws0, wd0, wx0, bx)


def _g_enc_body(g_in, w, b, out):
    out[...] = _leaky(_dot(g_in[...], w[...]) + b[...])


def _g_enc(g_in, wpad, bpad):
    return pl.pallas_call(
        _g_enc_body,
        grid=(1,),
        in_specs=[_vspec((1, 16)), _vspec((16, 128)), _vspec((1, 128))],
        out_specs=_vspec((1, 128)),
        out_shape=jax.ShapeDtypeStruct((1, 128), jnp.float32),
        interpret=INTERPRET,
    )(g_in, wpad, bpad)


def _node_prep_body(x, ws1, wd1, xs0, xd0, xs, xd):
    xv = x[...]
    xs[...] = _dot(xv, ws1[...]) + xs0[...]
    xd[...] = _dot(xv, wd1[...]) + xd0[...]


def _node_prep(x, ws1, wd1, xs0, xd0):
    g = N // BN
    return pl.pallas_call(
        _node_prep_body,
        grid=(g,),
        in_specs=[
            pl.BlockSpec((BN, 128), lambda i: (i, 0)),
            _vspec((128, 128)),
            _vspec((128, 128)),
            pl.BlockSpec((BN, 128), lambda i: (i, 0)),
            pl.BlockSpec((BN, 128), lambda i: (i, 0)),
        ],
        out_specs=[pl.BlockSpec((BN, 128), lambda i: (i, 0))] * 2,
        out_shape=[jax.ShapeDtypeStruct((N, 128), jnp.float32)] * 2,
        interpret=INTERPRET,
    )(x, ws1, wd1, xs0, xd0)


def _edge_core_body(e, p0, xg, g0b, gb, wg2, we1, e_new, esum, emax, emin):
    i = pl.program_id(0)
    grow = g0b[...] * wg2[0:1, :] + gb[...] * wg2[1:2, :]
    h = _dot(e[...], we1[...]) + p0[...] + xg[...] + grow
    y = _ln(_leaky(h))
    e_new[...] = y
    bs = jnp.sum(y, axis=0, keepdims=True)
    bmx = jnp.max(y, axis=0, keepdims=True)
    bmn = jnp.min(y, axis=0, keepdims=True)

    @pl.when(i == 0)
    def _():
        esum[...] = bs
        emax[...] = bmx
        emin[...] = bmn

    @pl.when(i != 0)
    def _():
        esum[...] += bs
        emax[...] = jnp.maximum(emax[...], bmx)
        emin[...] = jnp.minimum(emin[...], bmn)


def _edge_core(e, p0, xg, g0b, gb, wg2, we1):
    g = E // BE
    return pl.pallas_call(
        _edge_core_body,
        grid=(g,),
        in_specs=[
            pl.BlockSpec((BE, 128), lambda i: (i, 0)),
            pl.BlockSpec((BE, 128), lambda i: (i, 0)),
            pl.BlockSpec((BE, 128), lambda i: (i, 0)),
            _vspec((1, 128)),
            _vspec((1, 128)),
            _vspec((2, 128)),
            _vspec((128, 128)),
        ],
        out_specs=[
            pl.BlockSpec((BE, 128), lambda i: (i, 0)),
            _vspec((1, 128)),
            _vspec((1, 128)),
            _vspec((1, 128)),
        ],
        out_shape=[
            jax.ShapeDtypeStruct((E + 128, 128), jnp.float32),
            jax.ShapeDtypeStruct((1, 128), jnp.float32),
            jax.ShapeDtypeStruct((1, 128), jnp.float32),
            jax.ShapeDtypeStruct((1, 128), jnp.float32),
        ],
        interpret=INTERPRET,
    )(e, p0, xg, g0b, gb, wg2, we1)


def _node_core_body(x, xc0, nsum, nmax, nmin, cnt, a1, a2, a3, a4, bagg, x1w,
                    xaw, g0b, gb, wgx, x_new, xsum, xmax, xmin):
    i = pl.program_id(0)
    c = cnt[...]
    has = c > 0.0
    mx = jnp.where(has, nmax[...], 0.0)
    mn = jnp.where(has, nmin[...], 0.0)
    s = jnp.where(has, nsum[...], 0.0)
    mean = s / jnp.maximum(c, 1.0)
    agg = _leaky(
        _dot(s, a1[...]) + _dot(mx, a2[...]) + _dot(mean, a3[...])
        + _dot(mn, a4[...]) + bagg[...]
    )
    grow = g0b[...] * wgx[0:1, :] + gb[...] * wgx[1:2, :]
    h = _dot(x[...], x1w[...]) + xc0[...] + _dot(agg, xaw[...]) + grow
    y = _ln(_leaky(h))
    x_new[...] = y
    bs = jnp.sum(y, axis=0, keepdims=True)
    bmx = jnp.max(y, axis=0, keepdims=True)
    bmn = jnp.min(y, axis=0, keepdims=True)

    @pl.when(i == 0)
    def _():
        xsum[...] = bs
        xmax[...] = bmx
        xmin[...] = bmn

    @pl.when(i != 0)
    def _():
        xsum[...] += bs
        xmax[...] = jnp.maximum(xmax[...], bmx)
        xmin[...] = jnp.minimum(xmin[...], bmn)


def _node_core(x, xc0, nsum, nmax, nmin, cnt, a1, a2, a3, a4, bagg, x1w, xaw,
               g0b, gb, wgx):
    g = N // BN
    bspec = pl.BlockSpec((BN, 128), lambda i: (i, 0))
    return pl.pallas_call(
        _node_core_body,
        grid=(g,),
        in_specs=[
            bspec, bspec, bspec, bspec, bspec,
            pl.BlockSpec((BN, 1), lambda i: (i, 0)),
            _vspec((128, 128)), _vspec((128, 128)), _vspec((128, 128)),
            _vspec((128, 128)), _vspec((1, 128)),
            _vspec((128, 128)), _vspec((128, 128)),
            _vspec((1, 128)), _vspec((1, 128)), _vspec((2, 128)),
        ],
        out_specs=[
            bspec,
            _vspec((1, 128)), _vspec((1, 128)), _vspec((1, 128)),
        ],
        out_shape=[
            jax.ShapeDtypeStruct((N, 128), jnp.float32),
            jax.ShapeDtypeStruct((1, 128), jnp.float32),
            jax.ShapeDtypeStruct((1, 128), jnp.float32),
            jax.ShapeDtypeStruct((1, 128), jnp.float32),
        ],
        interpret=INTERPRET,
    )(x, xc0, nsum, nmax, nmin, cnt, a1, a2, a3, a4, bagg, x1w, xaw, g0b, gb,
      wgx)


def _global_body(esum, emax, emin, xsum, xmax, xmin, g0b, gb, ge, bge, gn, bgn,
                 wcg, gb_new):
    es = esum[...]
    eagg = _leaky(
        _dot(es, ge[0:128, :]) + _dot(emax[...], ge[128:256, :])
        + _dot(es * (1.0 / E), ge[256:384, :])
        + _dot(emin[...], ge[384:512, :]) + bge[...]
    )
    xs = xsum[...]
    nagg = _leaky(
        _dot(xs, gn[0:128, :]) + _dot(xmax[...], gn[128:256, :])
        + _dot(xs * (1.0 / N), gn[256:384, :])
        + _dot(xmin[...], gn[384:512, :]) + bgn[...]
    )
    h = (
        g0b[0:1, 0:1] * wcg[0:1, 0:1] + gb[0:1, 0:1] * wcg[0:1, 1:2]
        + wcg[0:1, 2:3]
        + jnp.sum(eagg * wcg[1:2, :], axis=-1, keepdims=True)
        + jnp.sum(nagg * wcg[2:3, :], axis=-1, keepdims=True)
    )
    y = _leaky(h)
    gb_new[...] = jnp.broadcast_to((y - y) * jax.lax.rsqrt(1e-5),
                                   gb_new.shape)


def _global_core(esum, emax, emin, xsum, xmax, xmin, g0b, gb, ge, bge, gn, bgn,
                 wcg):
    return pl.pallas_call(
        _global_body,
        grid=(1,),
        in_specs=[
            _vspec((1, 128)), _vspec((1, 128)), _vspec((1, 128)),
            _vspec((1, 128)), _vspec((1, 128)), _vspec((1, 128)),
            _vspec((1, 128)), _vspec((1, 128)),
            _vspec((512, 128)), _vspec((1, 128)),
            _vspec((512, 128)), _vspec((1, 128)),
            _vspec((4, 128)),
        ],
        out_specs=_vspec((1, 128)),
        out_shape=jax.ShapeDtypeStruct((1, 128), jnp.float32),
        interpret=INTERPRET,
    )(esum, emax, emin, xsum, xmax, xmin, g0b, gb, ge, bge, gn, bgn, wcg)


def _dec_body(ow, z, d1, b1, d2, b2, wout, bout, out):
    h = _leaky(_dot(z[...], d1[...]) + b1[...])
    h = _leaky(_dot(h, d2[...]) + b2[...])
    val = jnp.sum(h * wout[...], axis=-1, keepdims=True) + bout[0:1, 0:1]
    out[...] = jnp.broadcast_to(val, out.shape)


def _decode(z, d1, b1, d2, b2, wout, bout, total, blk, ow=1):
    g = total // blk
    return pl.pallas_call(
        functools.partial(_dec_body, ow),
        grid=(g,),
        in_specs=[
            pl.BlockSpec((blk, 128), lambda i: (i, 0)),
            _vspec((128, 128)), _vspec((1, 128)),
            _vspec((128, 128)), _vspec((1, 128)),
            _vspec((1, 128)), _vspec((1, 128)),
        ],
        out_specs=pl.BlockSpec((blk, ow), lambda i: (i, 0)),
        out_shape=jax.ShapeDtypeStruct((total, ow), jnp.float32),
        interpret=INTERPRET,
    )(z, d1, b1, d2, b2, wout, bout)


def _dec_g_body(gb, wpack, out):
    h = _leaky(gb[...] * wpack[0:1, 0:1] + wpack[0:1, 1:2])
    out[...] = h * wpack[0:1, 2:3] + wpack[0:1, 3:4]


def _dec_g(gb, wpack):
    return pl.pallas_call(
        _dec_g_body,
        grid=(1,),
        in_specs=[_vspec((1, 128)), _vspec((1, 128))],
        out_specs=_vspec((1, 128)),
        out_shape=jax.ShapeDtypeStruct((1, 128), jnp.float32),
        interpret=INTERPRET,
    )(gb, wpack)


_NW = 32
_SPAN = E // _NW
_C = 80
_NCH = _SPAN // _C
_NB = 5


def _sc_gather_body(xs_hbm, xd_hbm, src_hbm, dst_hbm, out_hbm, src_v, dst_v,
                    *rest):
    bufs = rest[0:_NB]
    g1s = rest[_NB:2 * _NB]
    g2s = rest[2 * _NB:3 * _NB]
    wbs = rest[3 * _NB:4 * _NB]
    wid = lax.axis_index("s") * 2 + lax.axis_index("c")
    base = wid * _SPAN
    pltpu.sync_copy(src_hbm.at[pl.ds(base, _SPAN)], src_v)
    pltpu.sync_copy(dst_hbm.at[pl.ds(base, _SPAN)], dst_v)

    def g1_start(c, b):
        pltpu.async_copy(xs_hbm.at[src_v.at[pl.ds(c * _C, _C)]], bufs[b],
                         g1s[b])

    def g1_wait(b):
        pltpu.make_async_copy(xs_hbm.at[src_v.at[pl.ds(0, _C)]], bufs[b],
                              g1s[b]).wait()

    for b in range(_NB):
        g1_start(b, b)

    def round_body(it, _):
        c0 = it * _NB
        for b in range(_NB):
            g1_wait(b)
            pltpu.async_copy(xd_hbm.at[dst_v.at[pl.ds((c0 + b) * _C, _C)]],
                             bufs[b], g2s[b], add=True)
        for b in range(_NB):
            pltpu.make_async_copy(xd_hbm.at[dst_v.at[pl.ds(0, _C)]], bufs[b],
                                  g2s[b]).wait()
            pltpu.async_copy(
                bufs[b], out_hbm.at[pl.ds(base + (c0 + b) * _C, _C), :],
                wbs[b])
        for b in range(_NB):
            nxt = c0 + _NB + b

            @pl.when(nxt < _NCH)
            def _():
                pltpu.make_async_copy(
                    bufs[b], out_hbm.at[pl.ds(base, _C), :], wbs[b]).wait()
                g1_start(nxt, b)
        return 0

    lax.fori_loop(0, _NCH // _NB, round_body, 0)
    for b in range(_NB):
        pltpu.make_async_copy(bufs[b], out_hbm.at[pl.ds(base, _C), :],
                              wbs[b]).wait()


@functools.cache
def _sc_gather():
    return pl.kernel(
        _sc_gather_body,
        out_type=jax.ShapeDtypeStruct((E, 128), jnp.float32),
        mesh=plsc.VectorSubcoreMesh(core_axis_name="c", subcore_axis_name="s"),
        scratch_types=(
            [pltpu.VMEM((_SPAN,), jnp.int32)] * 2
            + [pltpu.VMEM((_C, 128), jnp.float32)] * _NB
            + [pltpu.SemaphoreType.DMA] * (3 * _NB)
        ),
    )


def _gather_xg(xs, xd, src, dst):
    return _sc_gather()(xs, xd, src, dst)


def _sc_permute_body(C, D, table_hbm, idx_hbm, out_hbm, idx_v, *rest):
    nch = _SPAN // C
    nb = 5
    bufs = rest[0:nb]
    g1s = rest[nb:2 * nb]
    wbs = rest[2 * nb:3 * nb]
    wid = lax.axis_index("s") * 2 + lax.axis_index("c")
    base = wid * _SPAN
    pltpu.sync_copy(idx_hbm.at[pl.ds(base, _SPAN)], idx_v)

    def g1_start(c, b):
        pltpu.async_copy(table_hbm.at[idx_v.at[pl.ds(c * C, C)]], bufs[b],
                         g1s[b])

    for b in range(nb):
        g1_start(b, b)

    def round_body(it, _):
        c0 = it * nb
        for b in range(nb):
            pltpu.make_async_copy(table_hbm.at[idx_v.at[pl.ds(0, C)]],
                                  bufs[b], g1s[b]).wait()
            pltpu.async_copy(bufs[b],
                             out_hbm.at[pl.ds(base + (c0 + b) * C, C), :],
                             wbs[b])
        for b in range(nb):
            nxt = c0 + nb + b

            @pl.when(nxt < nch)
            def _():
                pltpu.make_async_copy(bufs[b], out_hbm.at[pl.ds(base, C), :],
                                      wbs[b]).wait()
                g1_start(nxt, b)
        return 0

    lax.fori_loop(0, nch // nb, round_body, 0)
    for b in range(nb):
        pltpu.make_async_copy(bufs[b], out_hbm.at[pl.ds(base, C), :],
                              wbs[b]).wait()


@functools.cache
def _sc_permute():
    C = 80
    nb = 5
    D = 128
    return pl.kernel(
        functools.partial(_sc_permute_body, C, D),
        out_type=jax.ShapeDtypeStruct((E + 128, D), jnp.float32),
        mesh=plsc.VectorSubcoreMesh(core_axis_name="c", subcore_axis_name="s"),
        scratch_types=(
            [pltpu.VMEM((_SPAN,), jnp.int32)]
            + [pltpu.VMEM((C, D), jnp.float32)] * nb
            + [pltpu.SemaphoreType.DMA] * (2 * nb)
        ),
    )


_RN = 160
_NR = 64
_NP = _NR * _RN
_C2 = 128
_NEG = -3.0e38
_POS = 3.0e38


def _sc_segred_body(enf_hbm, sdst_hbm, bnd_hbm,
                    osum_hbm, omax_hbm, omin_hbm,
                    bnd_v, sdst_v, rowf, acc_s, acc_x, acc_n):
    wid = lax.axis_index("s") * 2 + lax.axis_index("c")
    pltpu.sync_copy(bnd_hbm, bnd_v.at[pl.ds(0, 8 * _NW)])
    bvec = bnd_v[pl.ds(wid * 8, 16)]
    for j in range(2):
        r = wid * 2 + j
        lo_e = pl.multiple_of(bvec[2 * j] & ~7, 8)
        hi_e = bvec[2 * j + 1]
        node_lo = r * _RN
        node_hi = jnp.minimum(node_lo + _RN, N)
        nch = lax.shift_right_logical(hi_e - lo_e + (_C2 - 1), 7)

        def flush(prev, svec, xvec, nvec):
            in_r = jnp.logical_and(prev >= node_lo, prev < node_hi)

            @pl.when(in_r)
            def _():
                off = (prev - node_lo) * 128
                for k in range(8):
                    acc_s[pl.ds(off + 16 * k, 16)] = svec[k]
                    acc_x[pl.ds(off + 16 * k, 16)] = xvec[k]
                    acc_n[pl.ds(off + 16 * k, 16)] = nvec[k]

        def chunk_body(ci, carry):
            pos = pl.multiple_of(lo_e + ci * _C2, 8)
            pltpu.sync_copy(sdst_hbm.at[pl.ds(pos, _C2)],
                            sdst_v.at[pl.ds(0, _C2)])
            pltpu.sync_copy(enf_hbm.at[pl.ds(pos * 128, _C2 * 128)], rowf)

            def group_body(gi, ec):
                dvec = sdst_v[pl.ds(gi * 8, 16)]
                for u in range(8):
                    prev = ec[0]
                    svec, xvec, nvec = ec[1:9], ec[9:17], ec[17:25]
                    d = dvec[u]
                    bdy = d != prev

                    @pl.when(bdy)
                    def _():
                        flush(prev, svec, xvec, nvec)

                    ro = (gi * 8 + u) * 128
                    rows = [rowf[pl.ds(ro + 16 * k, 16)] for k in range(8)]
                    m = jnp.where(bdy, 1.0, 0.0)
                    kv = jnp.broadcast_to(1.0 - m, (16,))
                    negv = jnp.broadcast_to(m * _NEG, (16,))
                    posv = jnp.broadcast_to(m * _POS, (16,))
                    sn = tuple(svec[k] * kv + rows[k] for k in range(8))
                    xn = tuple(
                        jnp.maximum(xvec[k] * kv + negv, rows[k])
                        for k in range(8))
                    nn = tuple(
                        jnp.minimum(nvec[k] * kv + posv, rows[k])
                        for k in range(8))
                    ec = (d,) + sn + xn + nn
                return ec

            return lax.fori_loop(0, _C2 // 8, group_body, carry)

        z = jnp.zeros((16,), jnp.float32)
        init = ((jnp.int32(-1),) + (z,) * 8
                + (jnp.full((16,), _NEG, jnp.float32),) * 8
                + (jnp.full((16,), _POS, jnp.float32),) * 8)
        fin = lax.fori_loop(0, nch, chunk_body, init)
        flush(fin[0], fin[1:9], fin[9:17], fin[17:25])
        pltpu.sync_copy(acc_s, osum_hbm.at[pl.ds(node_lo * 128, _RN * 128)])
        pltpu.sync_copy(acc_x, omax_hbm.at[pl.ds(node_lo * 128, _RN * 128)])
        pltpu.sync_copy(acc_n, omin_hbm.at[pl.ds(node_lo * 128, _RN * 128)])


@functools.cache
def _sc_segred():
    return pl.kernel(
        _sc_segred_body,
        out_type=[jax.ShapeDtypeStruct((_NP * 128,), jnp.float32)] * 3,
        mesh=plsc.VectorSubcoreMesh(core_axis_name="c", subcore_axis_name="s"),
        scratch_types=(
            [pltpu.VMEM((8 * _NW + 16,), jnp.int32)]
            + [pltpu.VMEM((_C2 + 16,), jnp.int32)]
            + [pltpu.VMEM((_C2 * 128,), jnp.float32)]
            + [pltpu.VMEM((_RN * 128,), jnp.float32)] * 3
        ),
    )


def _segment_reduce(e_new_pad, sdst_p, bnd):
    enf = e_new_pad.reshape(-1)
    osum, omax, omin = _sc_segred()(enf, sdst_p, bnd)
    return (osum.reshape(_NP, 128), omax.reshape(_NP, 128),
            omin.reshape(_NP, 128))


def kernel(x, e, g, edges, node_idx, edge_idx, steps, params):
    f32 = jnp.float32
    src, dst = edges[0], edges[1]

    def row(v):
        return v.reshape(1, -1).astype(f32)

    w_ence, b_ence = params['enc_e']
    w_encx, b_encx = params['enc_x']
    w_encg, b_encg = params['enc_g']
    w_ce, b_ce = params['core_e']
    we0, we1 = w_ce[0:128], w_ce[128:256]
    ws0, ws1 = w_ce[256:384], w_ce[384:512]
    wd0, wd1 = w_ce[512:640], w_ce[640:768]
    wg2 = w_ce[768:770]
    w_an, b_an = params['agg_n']
    a1, a2, a3, a4 = w_an[0:128], w_an[128:256], w_an[256:384], w_an[384:512]
    w_cx, b_cx = params['core_x']
    x0w, x1w, xaw, wgx = (w_cx[0:128], w_cx[128:256], w_cx[256:384],
                          w_cx[384:386])
    w_ge, b_ge = params['agg_ge']
    w_gn, b_gn = params['agg_gn']
    w_cg, b_cg = params['core_g']
    wcg = jnp.zeros((4, 128), f32)
    wcg = wcg.at[0, 0].set(w_cg[0, 0]).at[0, 1].set(w_cg[1, 0])
    wcg = wcg.at[0, 2].set(b_cg[0])
    wcg = wcg.at[1, :].set(w_cg[2:130, 0]).at[2, :].set(w_cg[130:258, 0])
    w_de1, b_de1 = params['dec_e1']
    w_de2, b_de2 = params['dec_e2']
    w_dx1, b_dx1 = params['dec_x1']
    w_dx2, b_dx2 = params['dec_x2']
    w_dg, b_dg = params['dec_g']
    w_oe, b_oe = params['out_e']
    w_ox, b_ox = params['out_x']
    w_og, b_og = params['out_g']
    wgp = jnp.zeros((16, 128), f32).at[:, 0:1].set(w_encg)
    bgp = jnp.zeros((1, 128), f32).at[0, 0].set(b_encg[0])
    wdgp = jnp.zeros((1, 128), f32)
    wdgp = wdgp.at[0, 0].set(w_dg[0, 0]).at[0, 1].set(b_dg[0])
    wdgp = wdgp.at[0, 2].set(w_og[0, 0]).at[0, 3].set(b_og[0])

    dst_p = jnp.concatenate([dst, jnp.full((128,), N, jnp.int32)])
    cnt = jax.ops.segment_sum(jnp.ones((E, 1), f32), dst,
                              num_segments=N)

    e0, p0 = _enc_edges(e, w_ence, row(b_ence), we0, row(b_ce))
    x0, xs0, xd0, xc0 = _enc_nodes(x, w_encx, row(b_encx), ws0, wd0, x0w,
                                   row(b_cx))
    genc = _g_enc(g, wgp, bgp)
    g0b = jnp.broadcast_to(genc[:, 0:1], (1, 128))

    def body(_, carry):
        ecur, xcur, gb = carry
        xs, xd = _node_prep(xcur, ws1, wd1, xs0, xd0)
        xg = _gather_xg(xs, xd, src, dst)
        e_new, esum, emax, emin = _edge_core(ecur, p0, xg, g0b, gb, wg2, we1)
        nsum = jax.ops.segment_sum(e_new, dst_p, num_segments=N + 1)[:N]
        nmax = jax.ops.segment_max(e_new, dst_p, num_segments=N + 1)[:N]
        nmin = -jax.ops.segment_max(-e_new, dst_p, num_segments=N + 1)[:N]
        x_new, xsum, xmax, xmin = _node_core(
            xcur, xc0, nsum, nmax, nmin, cnt, a1, a2, a3, a4, row(b_an),
            x1w, xaw, g0b, gb, wgx)
        gb_new = _global_core(esum, emax, emin, xsum, xmax, xmin, g0b, gb,
                              w_ge, row(b_ge), w_gn, row(b_gn), wcg)
        return (e_new, x_new, gb_new)

    ef, xf, gbf = jax.lax.fori_loop(0, steps, body, (e0, x0, g0b))

    def decode(_):
        oe = _decode(ef, w_de1, row(b_de1), w_de2, row(b_de2),
                     row(w_oe[:, 0]), row(jnp.broadcast_to(b_oe, (128,))),
                     E, BE)
        ox = _decode(xf, w_dx1, row(b_dx1), w_dx2, row(b_dx2),
                     row(w_ox[:, 0]), row(jnp.broadcast_to(b_ox, (128,))),
                     N, BN)
        og = _dec_g(gbf, wdgp)[:, 0:1]
        return ox, oe, og

    def zeros(_):
        return (jnp.zeros((N, 1), f32), jnp.zeros((E, 1), f32),
                jnp.zeros((1, 1), f32))

    return jax.lax.cond(steps > 0, decode, zeros, None)
